# baseline (speedup 1.0000x reference)
import functools

import jax
import jax.numpy as jnp
from jax import lax
from jax.experimental import pallas as pl
from jax.experimental.pallas import tpu as pltpu
from jax.experimental.pallas import tpu_sc as plsc

N = 200000
C = 64
K = 20
VOXEL_SIZE = 0.02
BLOCK = 20480

_NW = 32
_SC_CHUNK = 6256
_SC_LAST = N - (_NW - 1) * _SC_CHUNK
_CNT_OFF = 24


def _fused_kernel(feat_ref, coord_ref, w1_ref, b1_ref, w2_ref, b2_ref,
                  wseg_ref, bseg_ref, pow_ref,
                  logit_ref, center_ref, seg_ref, w_ref):
    i = pl.program_id(0)

    feat = feat_ref[...]

    logits = jnp.dot(wseg_ref[...], feat, preferred_element_type=jnp.float32)
    logits = logits + bseg_ref[...]
    logit_ref[...] = logits

    h = jnp.dot(w1_ref[...], feat, preferred_element_type=jnp.float32)
    h = jnp.maximum(h + b1_ref[...], 0.0)
    bias = jnp.dot(w2_ref[...], h, preferred_element_type=jnp.float32)
    center_ref[...] = coord_ref[...] * (1.0 / VOXEL_SIZE) + (bias + b2_ref[...])

    colmax = jnp.max(logits, axis=0, keepdims=True)
    is_max = (logits == colmax).astype(jnp.float32)
    exps = jnp.exp(logits - colmax)
    packed = jnp.sum(is_max * pow_ref[...], axis=0, keepdims=True)
    expo = (jax.lax.bitcast_convert_type(packed, jnp.int32) >> 23) - 127
    idx = (K - 1) - expo
    seg_ref[...] = idx.reshape(-1)

    conf = 1.0 / jnp.sum(exps, axis=0, keepdims=True)
    col = jax.lax.broadcasted_iota(jnp.int32, idx.shape, 1) + i * BLOCK
    maskb = jnp.logical_and(idx >= 2, col < N)
    w_ref[...] = jnp.where(maskb, conf, 0.0).reshape(-1)


@functools.partial(
    pl.kernel,
    mesh=plsc.VectorSubcoreMesh(core_axis_name="c", subcore_axis_name="s"),
    out_type=jax.ShapeDtypeStruct((_NW, 48, 16), jnp.float32),
    scratch_types=[
        pltpu.VMEM((_SC_CHUNK,), jnp.int32),
        pltpu.VMEM((_SC_CHUNK,), jnp.float32),
        pltpu.VMEM((48, 16), jnp.float32),
    ],
)
def _sc_bins(seg_hbm, w_hbm, out_hbm, seg_v, w_v, bins_v):
    wid = lax.axis_index("s") * 2 + lax.axis_index("c")
    base = wid * _SC_CHUNK

    zero = jnp.zeros((16,), jnp.float32)
    zi = jnp.zeros((16,), jnp.int32)
    for r in range(48):
        bins_v[r, :] = zero

    @pl.when(wid < _NW - 1)
    def _full():
        pltpu.sync_copy(seg_hbm.at[pl.ds(base, _SC_CHUNK)], seg_v)
        pltpu.sync_copy(w_hbm.at[pl.ds(base, _SC_CHUNK)], w_v)

    @pl.when(wid == _NW - 1)
    def _last():
        pltpu.sync_copy(seg_hbm.at[pl.ds(base, _SC_LAST)],
                        seg_v.at[pl.ds(0, _SC_LAST)])
        pltpu.sync_copy(w_hbm.at[pl.ds(base, _SC_LAST)],
                        w_v.at[pl.ds(0, _SC_LAST)])
        for t in range(_SC_LAST // 16, _SC_CHUNK // 16):
            seg_v[pl.ds(t * 16, 16)] = zi
            w_v[pl.ds(t * 16, 16)] = zero

    def body(j, carry):
        sc, cn = carry
        sv = seg_v[pl.ds(j * 16, 16)]
        wv = w_v[pl.ds(j * 16, 16)]
        sc2 = []
        cn2 = []
        for k in range(K):
            eq = sv == k
            sc2.append(sc[k] + jnp.where(eq, wv, 0.0))
            if k >= 2:
                cn2.append(cn[k - 2] + jnp.where(eq, 1.0, 0.0))
        return sc2, cn2

    init = ([zero] * K, [zero] * (K - 2))
    sc, cn = lax.fori_loop(0, _SC_CHUNK // 16, body, init)
    for k in range(K):
        bins_v[k, :] = sc[k]
    for k in range(2, K):
        bins_v[_CNT_OFF + k, :] = cn[k - 2]

    pltpu.sync_copy(bins_v, out_hbm.at[wid])


@jax.jit
def kernel(feat, coord, W1, b1, gamma, beta, rmean, rvar, W2, b2, Wseg, bseg):
    scale = gamma * jax.lax.rsqrt(rvar + 1e-3)
    w1t = W1.T * scale[:, None]
    b1t = ((b1 - rmean) * scale + beta)[:, None]
    w2t = W2.T * (1.0 / VOXEL_SIZE)
    b2t = (b2 * (1.0 / VOXEL_SIZE))[:, None]

    pw = jnp.asarray([float(1 << (K - 1 - j)) for j in range(K)],
                     jnp.float32)[:, None]

    grid = (N + BLOCK - 1) // BLOCK
    out_shape = (
        jax.ShapeDtypeStruct((K, N), jnp.float32),
        jax.ShapeDtypeStruct((3, N), jnp.float32),
        jax.ShapeDtypeStruct((N,), jnp.int32),
        jax.ShapeDtypeStruct((N,), jnp.float32),
    )
    logit_t, center_t, seg_t, w_t = pl.pallas_call(
        _fused_kernel,
        grid=(grid,),
        in_specs=[
            pl.BlockSpec((C, BLOCK), lambda i: (0, i)),
            pl.BlockSpec((3, BLOCK), lambda i: (0, i)),
            pl.BlockSpec((C, C), lambda i: (0, 0)),
            pl.BlockSpec((C, 1), lambda i: (0, 0)),
            pl.BlockSpec((3, C), lambda i: (0, 0)),
            pl.BlockSpec((3, 1), lambda i: (0, 0)),
            pl.BlockSpec((K, C), lambda i: (0, 0)),
            pl.BlockSpec((K, 1), lambda i: (0, 0)),
            pl.BlockSpec((K, 1), lambda i: (0, 0)),
        ],
        out_specs=[
            pl.BlockSpec((K, BLOCK), lambda i: (0, i)),
            pl.BlockSpec((3, BLOCK), lambda i: (0, i)),
            pl.BlockSpec((BLOCK,), lambda i: (i,)),
            pl.BlockSpec((BLOCK,), lambda i: (i,)),
        ],
        out_shape=out_shape,
        compiler_params=pltpu.CompilerParams(
            dimension_semantics=("arbitrary",),
        ),
    )(feat.T, coord.T, w1t, b1t, w2t, b2t, Wseg.T, bseg[:, None], pw)

    partials = _sc_bins(seg_t, w_t)
    sums = jnp.sum(partials, axis=(0, 2))
    mean_conf = sums[:K] / (sums[_CNT_OFF:_CNT_OFF + K] + 1e-8)

    return logit_t.T, center_t.T, seg_t, mean_conf


# speedup vs baseline: 1.5410x; 1.5410x over previous
import jax
import jax.numpy as jnp
from jax.experimental import pallas as pl
from jax.experimental.pallas import tpu as pltpu

N = 200000
C = 64
K = 20
VOXEL_SIZE = 0.02
BLOCK = 20480


def _fused_kernel(feat_ref, coord_ref, w1_ref, b1_ref, w2_ref, b2_ref,
                  wseg_ref, bseg_ref, pow_ref,
                  logit_ref, center_ref, seg_ref, mean_ref,
                  s_acc, c_acc):
    i = pl.program_id(0)

    feat = feat_ref[...]

    logits = jnp.dot(wseg_ref[...], feat, preferred_element_type=jnp.float32)
    logits = logits + bseg_ref[...]
    logit_ref[...] = logits

    h = jnp.dot(w1_ref[...], feat, preferred_element_type=jnp.float32)
    h = jnp.maximum(h + b1_ref[...], 0.0)
    bias = jnp.dot(w2_ref[...], h, preferred_element_type=jnp.float32)
    center_ref[...] = coord_ref[...] * (1.0 / VOXEL_SIZE) + (bias + b2_ref[...])

    colmax = jnp.max(logits, axis=0, keepdims=True)
    is_max = (logits == colmax).astype(jnp.float32)
    exps = jnp.exp(logits - colmax)
    packed = jnp.sum(is_max * pow_ref[...], axis=0, keepdims=True)
    expo = (jax.lax.bitcast_convert_type(packed, jnp.int32) >> 23) - 127
    idx = (K - 1) - expo
    seg_ref[...] = idx.reshape(-1)

    conf = 1.0 / jnp.sum(exps, axis=0, keepdims=True)
    col = jax.lax.broadcasted_iota(jnp.int32, idx.shape, 1) + i * BLOCK
    maskb = jnp.logical_and(idx >= 2, col < N)
    w = jnp.where(maskb, conf, 0.0)
    cnt = jnp.where(maskb, 1.0, 0.0)
    sub = jax.lax.broadcasted_iota(jnp.int32, logits.shape, 0)
    onehot = (sub == idx).astype(jnp.float32)
    scores = jnp.sum(onehot * w, axis=1, keepdims=True)
    counts = jnp.sum(onehot * cnt, axis=1, keepdims=True)

    prev_s = jnp.where(i == 0, jnp.zeros_like(scores), s_acc[...])
    prev_c = jnp.where(i == 0, jnp.zeros_like(counts), c_acc[...])
    s = prev_s + scores
    c = prev_c + counts
    s_acc[...] = s
    c_acc[...] = c
    mean_ref[...] = s / (c + 1e-8)


@jax.jit
def kernel(feat, coord, W1, b1, gamma, beta, rmean, rvar, W2, b2, Wseg, bseg):
    scale = gamma * jax.lax.rsqrt(rvar + 1e-3)
    w1t = W1.T * scale[:, None]
    b1t = ((b1 - rmean) * scale + beta)[:, None]
    w2t = W2.T * (1.0 / VOXEL_SIZE)
    b2t = (b2 * (1.0 / VOXEL_SIZE))[:, None]

    pw = jnp.asarray([float(1 << (K - 1 - j)) for j in range(K)],
                     jnp.float32)[:, None]

    grid = (N + BLOCK - 1) // BLOCK
    out_shape = (
        jax.ShapeDtypeStruct((K, N), jnp.float32),
        jax.ShapeDtypeStruct((3, N), jnp.float32),
        jax.ShapeDtypeStruct((N,), jnp.int32),
        jax.ShapeDtypeStruct((K, 1), jnp.float32),
    )
    logit_t, center_t, seg_t, mean2d = pl.pallas_call(
        _fused_kernel,
        grid=(grid,),
        in_specs=[
            pl.BlockSpec((C, BLOCK), lambda i: (0, i)),
            pl.BlockSpec((3, BLOCK), lambda i: (0, i)),
            pl.BlockSpec((C, C), lambda i: (0, 0)),
            pl.BlockSpec((C, 1), lambda i: (0, 0)),
            pl.BlockSpec((3, C), lambda i: (0, 0)),
            pl.BlockSpec((3, 1), lambda i: (0, 0)),
            pl.BlockSpec((K, C), lambda i: (0, 0)),
            pl.BlockSpec((K, 1), lambda i: (0, 0)),
            pl.BlockSpec((K, 1), lambda i: (0, 0)),
        ],
        out_specs=[
            pl.BlockSpec((K, BLOCK), lambda i: (0, i)),
            pl.BlockSpec((3, BLOCK), lambda i: (0, i)),
            pl.BlockSpec((BLOCK,), lambda i: (i,)),
            pl.BlockSpec((K, 1), lambda i: (0, 0)),
        ],
        out_shape=out_shape,
        scratch_shapes=[
            pltpu.VMEM((K, 1), jnp.float32),
            pltpu.VMEM((K, 1), jnp.float32),
        ],
        compiler_params=pltpu.CompilerParams(
            dimension_semantics=("arbitrary",),
        ),
    )(feat.T, coord.T, w1t, b1t, w2t, b2t, Wseg.T, bseg[:, None], pw)

    return logit_t.T, center_t.T, seg_t, mean2d[:, 0]
